# baseline (device time: 49450 ns/iter reference)
import jax
import jax.numpy as jnp
from jax import lax
from jax.experimental import pallas as pl
from jax.experimental.pallas import tpu as pltpu

N_DEV = 8


def kernel(x, router_W, route_idx, expert_W):
    n_tok, d_model = x.shape
    e_loc, _, d_hid = expert_W.shape

    def body(x_ref, rw_ref, idx_ref, ew_ref, out_ref, *scratch):
        rs_bufs = [list(scratch[0:3]), list(scratch[3:6]), list(scratch[6:9])]
        rs_sbufs = [list(scratch[9:12]), list(scratch[12:15]), list(scratch[15:18])]
        ag_buf = scratch[18]
        send_sems, recv_sems = scratch[19], scratch[20]
        my = lax.axis_index("i")
        pz = my ^ 4
        py = my ^ 3
        px = my ^ 1
        s0 = (my >> 2) & 1
        s1 = (my >> 1) & 1
        s2 = (my & 1) ^ ((my >> 1) & 1)

        barrier_sem = pltpu.get_barrier_semaphore()
        for nbr in (pz, py, px):
            pl.semaphore_signal(
                barrier_sem, inc=1,
                device_id=(nbr,), device_id_type=pl.DeviceIdType.MESH,
            )

        xv = x_ref[...]
        scores = jnp.dot(xv, rw_ref[...], preferred_element_type=jnp.float32)
        s_max = jnp.max(scores, axis=-1, keepdims=True)
        p = jnp.exp(scores - s_max)
        probs = p / jnp.sum(p, axis=-1, keepdims=True)
        e0 = idx_ref[:, 0:1]
        e1 = idx_ref[:, 1:2]
        eids = lax.broadcasted_iota(jnp.int32, scores.shape, 1)
        g0 = jnp.sum(jnp.where(eids == e0, probs, 0.0), axis=-1, keepdims=True)
        g1 = jnp.sum(jnp.where(eids == e1, probs, 0.0), axis=-1, keepdims=True)
        gs = g0 + g1
        ws = []
        for l in range(e_loc):
            e_glob = my * e_loc + l
            ws.append((jnp.where(e0 == e_glob, g0, 0.0)
                       + jnp.where(e1 == e_glob, g1, 0.0)) / gs)

        pl.semaphore_wait(barrier_sem, 3)

        dims = {"z": (pz, s0), "y": (py, s1), "x": (px, s2)}
        parts = [
            {"base": 0, "size": 384, "order": ["z", "y", "x"]},
            {"base": 384, "size": 384, "order": ["y", "x", "z"]},
            {"base": 768, "size": 256, "order": ["x", "z", "y"]},
        ]
        cur_base = [jnp.int32(p["base"]) for p in parts]
        cur_size = [p["size"] for p in parts]

        def start_rs(j, step):
            prt, s = dims[parts[j]["order"][step]]
            half = cur_size[j] // 2
            keep = cur_base[j] + s * half
            send = cur_base[j] + (1 - s) * half
            rs_sbufs[j][step][...] = out_ref[pl.ds(send, half)].astype(
                jnp.bfloat16
            )
            rdma = pltpu.make_async_remote_copy(
                src_ref=rs_sbufs[j][step],
                dst_ref=rs_bufs[j][step],
                send_sem=send_sems.at[step * 3 + j],
                recv_sem=recv_sems.at[step * 3 + j],
                device_id=(prt,),
                device_id_type=pl.DeviceIdType.MESH,
            )
            rdma.start()
            return (rdma, keep, half, rs_bufs[j][step])

        def start_ag(j, step):
            prt, s = dims[parts[j]["order"][2 - step]]
            length = cur_size[j]
            a = cur_base[j]
            rdma = pltpu.make_async_remote_copy(
                src_ref=ag_buf.at[pl.ds(a, length)],
                dst_ref=ag_buf.at[pl.ds(a, length)],
                send_sem=send_sems.at[9 + step * 3 + j],
                recv_sem=recv_sems.at[9 + step * 3 + j],
                device_id=(prt,),
                device_id_type=pl.DeviceIdType.MESH,
            )
            rdma.start()
            return (rdma, s, length)

        pending = [None, None, None]
        for j, p in enumerate(parts):
            b, sz = p["base"], p["size"]
            xs = xv[b:b + sz]
            pp = jnp.zeros((sz, d_hid), jnp.float32)
            for l in range(e_loc):
                pp = pp + jnp.dot(
                    (ws[l][b:b + sz] * xs).astype(jnp.bfloat16),
                    ew_ref[l][...].astype(jnp.bfloat16),
                    preferred_element_type=jnp.float32,
                )
            out_ref[pl.ds(b, sz)] = pp
            pending[j] = start_rs(j, 0)

        for step in range(3):
            for j in range(3):
                rdma, keep, half, buf = pending[j]
                rdma.wait()
                out_ref[pl.ds(keep, half)] = (
                    out_ref[pl.ds(keep, half)] + buf[...].astype(jnp.float32)
                )
                cur_base[j] = keep
                cur_size[j] = half
                if step < 2:
                    pending[j] = start_rs(j, step + 1)
                else:
                    ag_buf[pl.ds(keep, half)] = out_ref[
                        pl.ds(keep, half)
                    ].astype(jnp.bfloat16)
                    pending[j] = start_ag(j, 0)

        for step in range(3):
            for j in range(3):
                rdma, s, length = pending[j]
                rdma.wait()
                cur_base[j] = cur_base[j] - s * length
                cur_size[j] = length * 2
                if step < 2:
                    pending[j] = start_ag(j, step + 1)
                else:
                    b, sz = parts[j]["base"], parts[j]["size"]
                    out_ref[pl.ds(b, sz)] = ag_buf[pl.ds(b, sz)].astype(
                        jnp.float32
                    )

    return pl.pallas_call(
        body,
        out_shape=jax.ShapeDtypeStruct((n_tok, d_hid), jnp.float32),
        in_specs=[
            pl.BlockSpec(memory_space=pltpu.VMEM),
            pl.BlockSpec(memory_space=pltpu.VMEM),
            pl.BlockSpec(memory_space=pltpu.VMEM),
            pl.BlockSpec(memory_space=pltpu.VMEM),
        ],
        out_specs=pl.BlockSpec(memory_space=pltpu.VMEM),
        scratch_shapes=[
            pltpu.VMEM((192, d_hid), jnp.bfloat16),
            pltpu.VMEM((96, d_hid), jnp.bfloat16),
            pltpu.VMEM((48, d_hid), jnp.bfloat16),
            pltpu.VMEM((192, d_hid), jnp.bfloat16),
            pltpu.VMEM((96, d_hid), jnp.bfloat16),
            pltpu.VMEM((48, d_hid), jnp.bfloat16),
            pltpu.VMEM((128, d_hid), jnp.bfloat16),
            pltpu.VMEM((64, d_hid), jnp.bfloat16),
            pltpu.VMEM((32, d_hid), jnp.bfloat16),
            pltpu.VMEM((192, d_hid), jnp.bfloat16),
            pltpu.VMEM((96, d_hid), jnp.bfloat16),
            pltpu.VMEM((48, d_hid), jnp.bfloat16),
            pltpu.VMEM((192, d_hid), jnp.bfloat16),
            pltpu.VMEM((96, d_hid), jnp.bfloat16),
            pltpu.VMEM((48, d_hid), jnp.bfloat16),
            pltpu.VMEM((128, d_hid), jnp.bfloat16),
            pltpu.VMEM((64, d_hid), jnp.bfloat16),
            pltpu.VMEM((32, d_hid), jnp.bfloat16),
            pltpu.VMEM((n_tok, d_hid), jnp.bfloat16),
            pltpu.SemaphoreType.DMA((18,)),
            pltpu.SemaphoreType.DMA((18,)),
        ],
        compiler_params=pltpu.CompilerParams(
            collective_id=0, vmem_limit_bytes=100 * 1024 * 1024
        ),
    )(x, router_W, route_idx, expert_W)


# device time: 49134 ns/iter; 1.0064x vs baseline; 1.0064x over previous
import jax
import jax.numpy as jnp
from jax import lax
from jax.experimental import pallas as pl
from jax.experimental.pallas import tpu as pltpu

N_DEV = 8


def kernel(x, router_W, route_idx, expert_W):
    n_tok, d_model = x.shape
    e_loc, _, d_hid = expert_W.shape

    def body(x_ref, rw_ref, idx_ref, ew_ref, out_ref, *scratch):
        rs_bufs = [list(scratch[0:3]), list(scratch[3:6]), list(scratch[6:9])]
        acc_buf = scratch[9]
        send_sems, recv_sems = scratch[10], scratch[11]
        my = lax.axis_index("i")
        pz = my ^ 4
        py = my ^ 3
        px = my ^ 1
        s0 = (my >> 2) & 1
        s1 = (my >> 1) & 1
        s2 = (my & 1) ^ ((my >> 1) & 1)

        barrier_sem = pltpu.get_barrier_semaphore()
        for nbr in (pz, py, px):
            pl.semaphore_signal(
                barrier_sem, inc=1,
                device_id=(nbr,), device_id_type=pl.DeviceIdType.MESH,
            )

        xv = x_ref[...]
        scores = jnp.dot(xv, rw_ref[...], preferred_element_type=jnp.float32)
        s_max = jnp.max(scores, axis=-1, keepdims=True)
        p = jnp.exp(scores - s_max)
        probs = p / jnp.sum(p, axis=-1, keepdims=True)
        e0 = idx_ref[:, 0:1]
        e1 = idx_ref[:, 1:2]
        eids = lax.broadcasted_iota(jnp.int32, scores.shape, 1)
        g0 = jnp.sum(jnp.where(eids == e0, probs, 0.0), axis=-1, keepdims=True)
        g1 = jnp.sum(jnp.where(eids == e1, probs, 0.0), axis=-1, keepdims=True)
        gs = g0 + g1
        ws = []
        for l in range(e_loc):
            e_glob = my * e_loc + l
            ws.append((jnp.where(e0 == e_glob, g0, 0.0)
                       + jnp.where(e1 == e_glob, g1, 0.0)) / gs)

        pl.semaphore_wait(barrier_sem, 3)

        dims = {"z": (pz, s0), "y": (py, s1), "x": (px, s2)}
        parts = [
            {"base": 0, "size": 384, "order": ["z", "y", "x"]},
            {"base": 384, "size": 384, "order": ["y", "x", "z"]},
            {"base": 768, "size": 256, "order": ["x", "z", "y"]},
        ]
        cur_base = [jnp.int32(p["base"]) for p in parts]
        cur_size = [p["size"] for p in parts]

        def start_rs(j, step):
            prt, s = dims[parts[j]["order"][step]]
            half = cur_size[j] // 2
            keep = cur_base[j] + s * half
            send = cur_base[j] + (1 - s) * half
            rdma = pltpu.make_async_remote_copy(
                src_ref=acc_buf.at[pl.ds(send, half)],
                dst_ref=rs_bufs[j][step],
                send_sem=send_sems.at[step * 3 + j],
                recv_sem=recv_sems.at[step * 3 + j],
                device_id=(prt,),
                device_id_type=pl.DeviceIdType.MESH,
            )
            rdma.start()
            return (rdma, keep, half, rs_bufs[j][step])

        def start_ag(j, step):
            prt, s = dims[parts[j]["order"][2 - step]]
            length = cur_size[j]
            a = cur_base[j]
            rdma = pltpu.make_async_remote_copy(
                src_ref=acc_buf.at[pl.ds(a, length)],
                dst_ref=acc_buf.at[pl.ds(a, length)],
                send_sem=send_sems.at[9 + step * 3 + j],
                recv_sem=recv_sems.at[9 + step * 3 + j],
                device_id=(prt,),
                device_id_type=pl.DeviceIdType.MESH,
            )
            rdma.start()
            return (rdma, s, length)

        pending = [None, None, None]
        for j, p in enumerate(parts):
            b, sz = p["base"], p["size"]
            xs = xv[b:b + sz]
            pp = jnp.zeros((sz, d_hid), jnp.float32)
            for l in range(e_loc):
                pp = pp + jnp.dot(
                    (ws[l][b:b + sz] * xs).astype(jnp.bfloat16),
                    ew_ref[l][...].astype(jnp.bfloat16),
                    preferred_element_type=jnp.float32,
                )
            acc_buf[pl.ds(b, sz)] = pp.astype(jnp.bfloat16)
            pending[j] = start_rs(j, 0)

        for step in range(3):
            for j in range(3):
                rdma, keep, half, buf = pending[j]
                rdma.wait()
                acc_buf[pl.ds(keep, half)] = (
                    acc_buf[pl.ds(keep, half)] + buf[...]
                )
                cur_base[j] = keep
                cur_size[j] = half
                pending[j] = start_rs(j, step + 1) if step < 2 else start_ag(j, 0)

        for step in range(3):
            for j in range(3):
                rdma, s, length = pending[j]
                rdma.wait()
                cur_base[j] = cur_base[j] - s * length
                cur_size[j] = length * 2
                if step < 2:
                    pending[j] = start_ag(j, step + 1)
                else:
                    b, sz = parts[j]["base"], parts[j]["size"]
                    out_ref[pl.ds(b, sz)] = acc_buf[pl.ds(b, sz)].astype(
                        jnp.float32
                    )

    return pl.pallas_call(
        body,
        out_shape=jax.ShapeDtypeStruct((n_tok, d_hid), jnp.float32),
        in_specs=[
            pl.BlockSpec(memory_space=pltpu.VMEM),
            pl.BlockSpec(memory_space=pltpu.VMEM),
            pl.BlockSpec(memory_space=pltpu.VMEM),
            pl.BlockSpec(memory_space=pltpu.VMEM),
        ],
        out_specs=pl.BlockSpec(memory_space=pltpu.VMEM),
        scratch_shapes=[
            pltpu.VMEM((192, d_hid), jnp.bfloat16),
            pltpu.VMEM((96, d_hid), jnp.bfloat16),
            pltpu.VMEM((48, d_hid), jnp.bfloat16),
            pltpu.VMEM((192, d_hid), jnp.bfloat16),
            pltpu.VMEM((96, d_hid), jnp.bfloat16),
            pltpu.VMEM((48, d_hid), jnp.bfloat16),
            pltpu.VMEM((128, d_hid), jnp.bfloat16),
            pltpu.VMEM((64, d_hid), jnp.bfloat16),
            pltpu.VMEM((32, d_hid), jnp.bfloat16),
            pltpu.VMEM((n_tok, d_hid), jnp.bfloat16),
            pltpu.SemaphoreType.DMA((18,)),
            pltpu.SemaphoreType.DMA((18,)),
        ],
        compiler_params=pltpu.CompilerParams(
            collective_id=0, vmem_limit_bytes=100 * 1024 * 1024
        ),
    )(x, router_W, route_idx, expert_W)


# device time: 46433 ns/iter; 1.0650x vs baseline; 1.0582x over previous
import jax
import jax.numpy as jnp
from jax import lax
from jax.experimental import pallas as pl
from jax.experimental.pallas import tpu as pltpu

N_DEV = 8


def kernel(x, router_W, route_idx, expert_W):
    n_tok, d_model = x.shape
    e_loc, _, d_hid = expert_W.shape

    N_PARTS = 6

    def body(x_ref, rw_ref, idx_ref, ew_ref, out_ref, *scratch):
        rs_bufs = [list(scratch[3 * j:3 * j + 3]) for j in range(N_PARTS)]
        acc_buf = scratch[3 * N_PARTS]
        send_sems, recv_sems = scratch[3 * N_PARTS + 1], scratch[3 * N_PARTS + 2]
        my = lax.axis_index("i")
        pz = my ^ 4
        py = my ^ 3
        px = my ^ 1
        s0 = (my >> 2) & 1
        s1 = (my >> 1) & 1
        s2 = (my & 1) ^ ((my >> 1) & 1)

        barrier_sem = pltpu.get_barrier_semaphore()
        for nbr in (pz, py, px):
            pl.semaphore_signal(
                barrier_sem, inc=1,
                device_id=(nbr,), device_id_type=pl.DeviceIdType.MESH,
            )

        xv = x_ref[...]
        scores = jnp.dot(xv, rw_ref[...], preferred_element_type=jnp.float32)
        s_max = jnp.max(scores, axis=-1, keepdims=True)
        p = jnp.exp(scores - s_max)
        probs = p / jnp.sum(p, axis=-1, keepdims=True)
        e0 = idx_ref[:, 0:1]
        e1 = idx_ref[:, 1:2]
        eids = lax.broadcasted_iota(jnp.int32, scores.shape, 1)
        g0 = jnp.sum(jnp.where(eids == e0, probs, 0.0), axis=-1, keepdims=True)
        g1 = jnp.sum(jnp.where(eids == e1, probs, 0.0), axis=-1, keepdims=True)
        gs = g0 + g1
        ws = []
        for l in range(e_loc):
            e_glob = my * e_loc + l
            ws.append((jnp.where(e0 == e_glob, g0, 0.0)
                       + jnp.where(e1 == e_glob, g1, 0.0)) / gs)

        pl.semaphore_wait(barrier_sem, 3)

        dims = {"z": (pz, s0), "y": (py, s1), "x": (px, s2)}
        parts = [
            {"base": 0, "size": 192, "order": ["z", "y", "x"]},
            {"base": 192, "size": 192, "order": ["y", "x", "z"]},
            {"base": 384, "size": 192, "order": ["x", "z", "y"]},
            {"base": 576, "size": 192, "order": ["z", "x", "y"]},
            {"base": 768, "size": 128, "order": ["x", "y", "z"]},
            {"base": 896, "size": 128, "order": ["y", "z", "x"]},
        ]
        cur_base = [jnp.int32(p["base"]) for p in parts]
        cur_size = [p["size"] for p in parts]

        def start_rs(j, step):
            prt, s = dims[parts[j]["order"][step]]
            half = cur_size[j] // 2
            keep = cur_base[j] + s * half
            send = cur_base[j] + (1 - s) * half
            rdma = pltpu.make_async_remote_copy(
                src_ref=acc_buf.at[pl.ds(send, half)],
                dst_ref=rs_bufs[j][step],
                send_sem=send_sems.at[step * 6 + j],
                recv_sem=recv_sems.at[step * 6 + j],
                device_id=(prt,),
                device_id_type=pl.DeviceIdType.MESH,
            )
            rdma.start()
            return (rdma, keep, half, rs_bufs[j][step])

        def start_ag(j, step):
            prt, s = dims[parts[j]["order"][2 - step]]
            length = cur_size[j]
            a = cur_base[j]
            rdma = pltpu.make_async_remote_copy(
                src_ref=acc_buf.at[pl.ds(a, length)],
                dst_ref=acc_buf.at[pl.ds(a, length)],
                send_sem=send_sems.at[18 + step * 6 + j],
                recv_sem=recv_sems.at[18 + step * 6 + j],
                device_id=(prt,),
                device_id_type=pl.DeviceIdType.MESH,
            )
            rdma.start()
            return (rdma, s, length)

        pending = [None] * N_PARTS
        for j, p in enumerate(parts):
            b, sz = p["base"], p["size"]
            xs = xv[b:b + sz]
            pp = jnp.zeros((sz, d_hid), jnp.float32)
            for l in range(e_loc):
                pp = pp + jnp.dot(
                    (ws[l][b:b + sz] * xs).astype(jnp.bfloat16),
                    ew_ref[l][...].astype(jnp.bfloat16),
                    preferred_element_type=jnp.float32,
                )
            acc_buf[pl.ds(b, sz)] = pp.astype(jnp.bfloat16)
            pending[j] = start_rs(j, 0)

        for step in range(3):
            for j in range(N_PARTS):
                rdma, keep, half, buf = pending[j]
                rdma.wait()
                acc_buf[pl.ds(keep, half)] = (
                    acc_buf[pl.ds(keep, half)] + buf[...]
                )
                cur_base[j] = keep
                cur_size[j] = half
                pending[j] = start_rs(j, step + 1) if step < 2 else start_ag(j, 0)

        for step in range(3):
            for j in range(N_PARTS):
                rdma, s, length = pending[j]
                rdma.wait()
                cur_base[j] = cur_base[j] - s * length
                cur_size[j] = length * 2
                if step < 2:
                    pending[j] = start_ag(j, step + 1)
                else:
                    b, sz = parts[j]["base"], parts[j]["size"]
                    out_ref[pl.ds(b, sz)] = acc_buf[pl.ds(b, sz)].astype(
                        jnp.float32
                    )

    return pl.pallas_call(
        body,
        out_shape=jax.ShapeDtypeStruct((n_tok, d_hid), jnp.float32),
        in_specs=[
            pl.BlockSpec(memory_space=pltpu.VMEM),
            pl.BlockSpec(memory_space=pltpu.VMEM),
            pl.BlockSpec(memory_space=pltpu.VMEM),
            pl.BlockSpec(memory_space=pltpu.VMEM),
        ],
        out_specs=pl.BlockSpec(memory_space=pltpu.VMEM),
        scratch_shapes=(
            [pltpu.VMEM((sz // (2 ** (k + 1)), d_hid), jnp.bfloat16)
             for sz in (192, 192, 192, 192, 128, 128) for k in range(3)]
            + [
                pltpu.VMEM((n_tok, d_hid), jnp.bfloat16),
                pltpu.SemaphoreType.DMA((36,)),
                pltpu.SemaphoreType.DMA((36,)),
            ]
        ),
        compiler_params=pltpu.CompilerParams(
            collective_id=0, vmem_limit_bytes=100 * 1024 * 1024
        ),
    )(x, router_W, route_idx, expert_W)
